# Initial kernel scaffold; baseline (speedup 1.0000x reference)
import jax
import jax.numpy as jnp
from jax.experimental import pallas as pl


def kernel(x, edge_index, batch, W1, b1, g1, be1, W2, b2, g2, be2, W3, b3, g3, be3, Wl, bl):
    raise NotImplementedError("write your pallas kernel here")


# speedup vs baseline: 2.9963x; 2.9963x over previous
import functools

import jax
import jax.numpy as jnp
from jax import lax
from jax.experimental import pallas as pl
from jax.experimental.pallas import tpu as pltpu
from jax.experimental.pallas import tpu_sc as plsc

G = 64
NC = 2
NS = 16
NT = NC * NS

ECH = 2000
CBUF = 1184
FLUSH = 1024
GCH = 128

_NEG_INF = float("-inf")


def _prep_body(nt_pad, n_edges, n_nodes, rb, edge_ref, csrc_ref, cdst_ref,
               cnt_ref, srcb, dstb, sb, db, cntb):
    c = lax.axis_index("c")
    s = lax.axis_index("s")
    t = s * NC + c
    lo = t * nt_pad
    hi = lo + nt_pad
    nvec = ECH // 16
    nch = n_edges // ECH

    def chunk_body(ci, carry):
        pltpu.sync_copy(edge_ref.at[pl.ds(ci * ECH, ECH)], srcb)
        pltpu.sync_copy(edge_ref.at[pl.ds(n_edges + ci * ECH, ECH)], dstb)

        def vec_body(vi, carry2):
            woff, flushed = carry2
            sv = srcb[pl.ds(vi * 16, 16)]
            dv = dstb[pl.ds(vi * 16, 16)]
            m = (dv >= lo) & (dv < hi)
            mi = m.astype(jnp.int32)
            csum = plsc.cumsum(mi)
            pos = woff + csum - 1
            plsc.store_scatter(sb, [pos], sv, mask=m)
            plsc.store_scatter(db, [pos], dv, mask=m)
            woff = woff + csum[15]
            do_flush = woff >= FLUSH

            @pl.when(do_flush)
            def _():
                off = pl.multiple_of(t * rb + flushed, 8)
                pltpu.sync_copy(sb.at[pl.ds(0, FLUSH)],
                                csrc_ref.at[pl.ds(off, FLUSH)])
                pltpu.sync_copy(db.at[pl.ds(0, FLUSH)],
                                cdst_ref.at[pl.ds(off, FLUSH)])
                sb[pl.ds(0, 16)] = sb[pl.ds(FLUSH, 16)]
                db[pl.ds(0, 16)] = db[pl.ds(FLUSH, 16)]

            woff = jnp.where(do_flush, woff - FLUSH, woff)
            flushed = jnp.where(do_flush, flushed + FLUSH, flushed)
            return woff, flushed

        return lax.fori_loop(0, nvec, vec_body, carry)

    woff, flushed = lax.fori_loop(0, nch, chunk_body,
                                  (jnp.int32(0), jnp.int32(0)))

    for k in range(8):
        sb[pl.ds(woff + k * 16, 16)] = jnp.full((16,), n_nodes, jnp.int32)
        db[pl.ds(woff + k * 16, 16)] = jnp.full((16,), 1, jnp.int32) * lo
    p8 = ((woff + GCH - 1) >> 7) << 7
    off = pl.multiple_of(t * rb + flushed, 8)
    pltpu.sync_copy(sb.at[pl.ds(0, CBUF - 32)],
                    csrc_ref.at[pl.ds(off, CBUF - 32)])
    pltpu.sync_copy(db.at[pl.ds(0, CBUF - 32)],
                    cdst_ref.at[pl.ds(off, CBUF - 32)])
    cntb[pl.ds(0, 16)] = jnp.full((16,), 1, jnp.int32) * (flushed + p8)
    pltpu.sync_copy(cntb, cnt_ref.at[pl.ds(pl.multiple_of(t * 16, 8), 16)])


def _make_prep(n_nodes, n_edges, nt_pad):
    rb = n_edges + CBUF + FLUSH
    rb = ((rb + 7) // 8) * 8
    mesh = plsc.VectorSubcoreMesh(core_axis_name="c", subcore_axis_name="s")
    return pl.kernel(
        functools.partial(_prep_body, nt_pad, n_edges, n_nodes, rb),
        out_type=[
            jax.ShapeDtypeStruct((NT * rb,), jnp.int32),
            jax.ShapeDtypeStruct((NT * rb,), jnp.int32),
            jax.ShapeDtypeStruct((NT * 16,), jnp.int32),
        ],
        mesh=mesh,
        compiler_params=pltpu.CompilerParams(needs_layout_passes=False),
        scratch_types=[
            pltpu.VMEM((ECH,), jnp.int32),
            pltpu.VMEM((ECH,), jnp.int32),
            pltpu.VMEM((CBUF,), jnp.int32),
            pltpu.VMEM((CBUF,), jnp.int32),
            pltpu.VMEM((16,), jnp.int32),
        ],
    )


def _seg_body(nt_pad, nh, rb, csrc_ref, cdst_ref, cnt_ref, b_ref, s_ref,
              sidx, dbuf, rows, acc, cntb, sem):
    c = lax.axis_index("c")
    s = lax.axis_index("s")
    t = s * NC + c
    lo = t * nt_pad
    nj = nh // 16

    pltpu.sync_copy(cnt_ref.at[pl.ds(pl.multiple_of(t * 16, 8), 16)], cntb)
    cnt = cntb[pl.ds(0, 16)][0]

    neg = jnp.full((16,), _NEG_INF, jnp.float32)

    def init_body(r, _):
        for j in range(nj):
            acc[r, pl.ds(j * 16, 16)] = neg
        return 0

    lax.fori_loop(0, nt_pad, init_body, 0)

    nchunks = cnt >> 7

    def chunk_body(ci, _):
        boff = pl.multiple_of(t * rb + ci * GCH, 8)
        pltpu.sync_copy(csrc_ref.at[pl.ds(boff, GCH)], sidx)
        pltpu.sync_copy(cdst_ref.at[pl.ds(boff, GCH)],
                        dbuf.at[pl.ds(0, GCH)])
        pltpu.async_copy(b_ref.at[sidx], rows, sem).wait()

        def edge_body(e, _):
            dl = dbuf[pl.ds(e, 16)][0] - lo
            for j in range(nj):
                sl = pl.ds(j * 16, 16)
                acc[dl, sl] = jnp.maximum(acc[dl, sl], rows[e, sl])
            return 0

        lax.fori_loop(0, GCH, edge_body, 0)
        return 0

    lax.fori_loop(0, nchunks, chunk_body, 0)
    pltpu.sync_copy(acc, s_ref.at[pl.ds(pl.multiple_of(lo, 8), nt_pad), :])


def _make_seg(nt_pad, nh, rb):
    mesh = plsc.VectorSubcoreMesh(core_axis_name="c", subcore_axis_name="s")
    return pl.kernel(
        functools.partial(_seg_body, nt_pad, nh, rb),
        out_type=[jax.ShapeDtypeStruct((NT * nt_pad, nh), jnp.float32)],
        mesh=mesh,
        compiler_params=pltpu.CompilerParams(needs_layout_passes=False),
        scratch_types=[
            pltpu.VMEM((GCH,), jnp.int32),
            pltpu.VMEM((GCH + 16,), jnp.int32),
            pltpu.VMEM((GCH, 128), jnp.float32),
            pltpu.VMEM((nt_pad, nh), jnp.float32),
            pltpu.VMEM((16,), jnp.int32),
            pltpu.SemaphoreType.DMA,
        ],
    )


def _dot(a, b_t):
    return lax.dot_general(a, b_t, (((1,), (1,)), ((), ())),
                           preferred_element_type=jnp.float32,
                           precision=lax.Precision.HIGHEST)


def _first_tc_body(n_nodes, d_in, x_ref, w_ref, b_ref, bout_ref, cout_ref):
    x = x_ref[...]
    w = w_ref[...]
    wa = w[:, :d_in]
    wb = w[:, d_in:]
    bmat = _dot(x, wb)
    cmat = _dot(x, wa - wb) + b_ref[...]
    bout_ref[pl.ds(0, n_nodes), :] = jnp.concatenate(
        [bmat, jnp.zeros_like(bmat)], axis=1)
    bout_ref[pl.ds(n_nodes, 8), :] = jnp.full((8, 2 * bmat.shape[1]), _NEG_INF,
                                              jnp.float32)
    cout_ref[...] = cmat


def _mid_tc_body(n_nodes, nh, c_ref, s_ref, g_ref, be_ref, w_ref, b_ref,
                 bout_ref, cout_ref):
    agg = jnp.maximum(c_ref[...] + s_ref[pl.ds(0, n_nodes), :], 0.0)
    mean = jnp.mean(agg, axis=0, keepdims=True)
    var = jnp.mean((agg - mean) ** 2, axis=0, keepdims=True)
    hn = (agg - mean) * lax.rsqrt(var + 1e-5) * g_ref[...] + be_ref[...]
    h = jnp.maximum(hn, 0.0)
    w = w_ref[...]
    wa = w[:, :nh]
    wb = w[:, nh:]
    bmat = _dot(h, wb)
    cmat = _dot(h, wa - wb) + b_ref[...]
    bout_ref[pl.ds(0, n_nodes), :] = jnp.concatenate(
        [bmat, jnp.zeros_like(bmat)], axis=1)
    bout_ref[pl.ds(n_nodes, 8), :] = jnp.full((8, 2 * bmat.shape[1]), _NEG_INF,
                                              jnp.float32)
    cout_ref[...] = cmat


def _final_tc_body(n_nodes, c_ref, s_ref, g_ref, be_ref, batch_ref, wl_ref,
                   bl_ref, out_ref):
    agg = jnp.maximum(c_ref[...] + s_ref[pl.ds(0, n_nodes), :], 0.0)
    mean = jnp.mean(agg, axis=0, keepdims=True)
    var = jnp.mean((agg - mean) ** 2, axis=0, keepdims=True)
    h = (agg - mean) * lax.rsqrt(var + 1e-5) * g_ref[...] + be_ref[...]
    gid = lax.broadcasted_iota(jnp.int32, (n_nodes, G), 1)
    oh = (batch_ref[...] == gid).astype(jnp.float32)
    sums = lax.dot_general(oh, h, (((0,), (0,)), ((), ())),
                           preferred_element_type=jnp.float32,
                           precision=lax.Precision.HIGHEST)
    counts = jnp.sum(oh, axis=0, keepdims=True)
    pooled = sums / jnp.maximum(counts.T, 1.0)
    out = _dot(pooled, wl_ref[...]) + bl_ref[...]
    m = jnp.max(out, axis=1, keepdims=True)
    e = jnp.exp(out - m)
    out_ref[...] = e / jnp.sum(e, axis=1, keepdims=True)


def kernel(x, edge_index, batch, W1, b1, g1, be1, W2, b2, g2, be2,
           W3, b3, g3, be3, Wl, bl):
    n, d = x.shape
    e = edge_index.shape[1]
    nh = W1.shape[0]
    nt_pad = ((n + NT - 1) // NT + 7) // 8 * 8

    prep = _make_prep(n, e, nt_pad)
    csrc, cdst, counts = prep(edge_index.reshape(-1))
    seg = _make_seg(nt_pad, nh, csrc.shape[0] // NT)

    b1r = b1.reshape(1, nh)
    b2r = b2.reshape(1, nh)
    b3r = b3.reshape(1, nh)
    g1r, be1r = g1.reshape(1, nh), be1.reshape(1, nh)
    g2r, be2r = g2.reshape(1, nh), be2.reshape(1, nh)
    g3r, be3r = g3.reshape(1, nh), be3.reshape(1, nh)
    blr = bl.reshape(1, -1)
    batch2 = batch.reshape(n, 1)

    first = pl.pallas_call(
        functools.partial(_first_tc_body, n, d),
        out_shape=[
            jax.ShapeDtypeStruct((n + 8, 2 * nh), jnp.float32),
            jax.ShapeDtypeStruct((n, nh), jnp.float32),
        ],
    )
    mid = pl.pallas_call(
        functools.partial(_mid_tc_body, n, nh),
        out_shape=[
            jax.ShapeDtypeStruct((n + 8, 2 * nh), jnp.float32),
            jax.ShapeDtypeStruct((n, nh), jnp.float32),
        ],
    )
    final = pl.pallas_call(
        functools.partial(_final_tc_body, n),
        out_shape=jax.ShapeDtypeStruct((G, Wl.shape[0]), jnp.float32),
    )

    B1, C1 = first(x, W1, b1r)
    (S1,) = seg(csrc, cdst, counts, B1)
    B2, C2 = mid(C1, S1, g1r, be1r, W2, b2r)
    (S2,) = seg(csrc, cdst, counts, B2)
    B3, C3 = mid(C2, S2, g2r, be2r, W3, b3r)
    (S3,) = seg(csrc, cdst, counts, B3)
    return final(C3, S3, g3r, be3r, batch2, Wl, blr)


# speedup vs baseline: 4.6665x; 1.5574x over previous
import functools

import jax
import jax.numpy as jnp
from jax import lax
from jax.experimental import pallas as pl
from jax.experimental.pallas import tpu as pltpu
from jax.experimental.pallas import tpu_sc as plsc

G = 64
NC = 2
NS = 16
NT = NC * NS

ECH = 3200
CBUF = 1344
FLUSH = 1024
GCH = 256

_NEG_INF = float("-inf")


def _prep_body(nt_pad, n_edges, n_nodes, rb, edge_ref, csrc_ref, cdst_ref,
               cnt_ref, srcb0, dstb0, srcb1, dstb1, sb, db, cntb, esem0, esem1):
    c = lax.axis_index("c")
    s = lax.axis_index("s")
    t = s * NC + c
    lo = t * nt_pad
    hi = lo + nt_pad
    nvec2 = ECH // 32
    nch = n_edges // ECH
    srcb = (srcb0, srcb1)
    dstb = (dstb0, dstb1)
    esem = (esem0, esem1)

    def fetch(ci, b):
        off = pl.multiple_of(ci * ECH, 8)
        pltpu.async_copy(edge_ref.at[pl.ds(off, ECH)], srcb[b], esem[b])
        off2 = pl.multiple_of(n_edges + ci * ECH, 8)
        pltpu.async_copy(edge_ref.at[pl.ds(off2, ECH)], dstb[b], esem[b])

    def wait_fetch(ci, b):
        off = pl.multiple_of(ci * ECH, 8)
        pltpu.make_async_copy(edge_ref.at[pl.ds(off, ECH)], srcb[b],
                              esem[b]).wait()
        off2 = pl.multiple_of(n_edges + ci * ECH, 8)
        pltpu.make_async_copy(edge_ref.at[pl.ds(off2, ECH)], dstb[b],
                              esem[b]).wait()

    fetch(0, 0)

    def pair_body(cp, carry):
        for bb in (0, 1):
            ci = 2 * cp + bb
            wait_fetch(ci, bb)

            @pl.when(ci + 1 < nch)
            def _(ci=ci, bb=bb):
                fetch(ci + 1, 1 - bb)

            def vec_body(vi, carry2, bb=bb):
                woff, flushed = carry2
                sv0 = srcb[bb][pl.ds(vi * 32, 16)]
                dv0 = dstb[bb][pl.ds(vi * 32, 16)]
                sv1 = srcb[bb][pl.ds(vi * 32 + 16, 16)]
                dv1 = dstb[bb][pl.ds(vi * 32 + 16, 16)]
                m0 = (dv0 >= lo) & (dv0 < hi)
                m1 = (dv1 >= lo) & (dv1 < hi)
                c0 = plsc.cumsum(m0.astype(jnp.int32))
                c1 = plsc.cumsum(m1.astype(jnp.int32))
                n0 = c0[15]
                pos0 = woff + c0 - 1
                pos1 = woff + n0 + c1 - 1
                plsc.store_scatter(sb, [pos0], sv0, mask=m0)
                plsc.store_scatter(db, [pos0], dv0, mask=m0)
                plsc.store_scatter(sb, [pos1], sv1, mask=m1)
                plsc.store_scatter(db, [pos1], dv1, mask=m1)
                woff = woff + n0 + c1[15]
                do_flush = woff >= FLUSH

                @pl.when(do_flush)
                def _():
                    off = pl.multiple_of(t * rb + flushed, 8)
                    pltpu.sync_copy(sb.at[pl.ds(0, FLUSH)],
                                    csrc_ref.at[pl.ds(off, FLUSH)])
                    pltpu.sync_copy(db.at[pl.ds(0, FLUSH)],
                                    cdst_ref.at[pl.ds(off, FLUSH)])
                    sb[pl.ds(0, 16)] = sb[pl.ds(FLUSH, 16)]
                    sb[pl.ds(16, 16)] = sb[pl.ds(FLUSH + 16, 16)]
                    db[pl.ds(0, 16)] = db[pl.ds(FLUSH, 16)]
                    db[pl.ds(16, 16)] = db[pl.ds(FLUSH + 16, 16)]

                woff = jnp.where(do_flush, woff - FLUSH, woff)
                flushed = jnp.where(do_flush, flushed + FLUSH, flushed)
                return woff, flushed

            carry = lax.fori_loop(0, nvec2, vec_body, carry)
        return carry

    woff, flushed = lax.fori_loop(0, nch // 2, pair_body,
                                  (jnp.int32(0), jnp.int32(0)))

    for k in range(GCH // 16):
        sb[pl.ds(woff + k * 16, 16)] = jnp.full((16,), n_nodes, jnp.int32)
        db[pl.ds(woff + k * 16, 16)] = jnp.full((16,), 1, jnp.int32) * lo
    p8 = ((woff + GCH - 1) >> 8) << 8
    off = pl.multiple_of(t * rb + flushed, 8)
    pltpu.sync_copy(sb.at[pl.ds(0, CBUF - 64)],
                    csrc_ref.at[pl.ds(off, CBUF - 64)])
    pltpu.sync_copy(db.at[pl.ds(0, CBUF - 64)],
                    cdst_ref.at[pl.ds(off, CBUF - 64)])
    cntb[pl.ds(0, 16)] = jnp.full((16,), 1, jnp.int32) * (flushed + p8)
    pltpu.sync_copy(cntb, cnt_ref.at[pl.ds(pl.multiple_of(t * 16, 8), 16)])


def _make_prep(n_nodes, n_edges, nt_pad):
    rb = n_edges + CBUF + FLUSH
    rb = ((rb + 7) // 8) * 8
    mesh = plsc.VectorSubcoreMesh(core_axis_name="c", subcore_axis_name="s")
    return pl.kernel(
        functools.partial(_prep_body, nt_pad, n_edges, n_nodes, rb),
        out_type=[
            jax.ShapeDtypeStruct((NT * rb,), jnp.int32),
            jax.ShapeDtypeStruct((NT * rb,), jnp.int32),
            jax.ShapeDtypeStruct((NT * 16,), jnp.int32),
        ],
        mesh=mesh,
        compiler_params=pltpu.CompilerParams(needs_layout_passes=False),
        scratch_types=[
            pltpu.VMEM((ECH,), jnp.int32),
            pltpu.VMEM((ECH,), jnp.int32),
            pltpu.VMEM((ECH,), jnp.int32),
            pltpu.VMEM((ECH,), jnp.int32),
            pltpu.VMEM((CBUF,), jnp.int32),
            pltpu.VMEM((CBUF,), jnp.int32),
            pltpu.VMEM((16,), jnp.int32),
            pltpu.SemaphoreType.DMA,
            pltpu.SemaphoreType.DMA,
        ],
    )


def _seg_body(nt_pad, nh, rb, csrc_ref, cdst_ref, cnt_ref, b_ref, s_ref,
              sidx0, sidx1, dbuf0, dbuf1, rows0, rows1, acc0, acc1, cntb,
              sem0, sem1):
    c = lax.axis_index("c")
    s = lax.axis_index("s")
    t = s * NC + c
    lo = t * nt_pad
    nj = nh // 16
    sidx = (sidx0, sidx1)
    dbuf = (dbuf0, dbuf1)
    rows = (rows0, rows1)
    sem = (sem0, sem1)

    pltpu.sync_copy(cnt_ref.at[pl.ds(pl.multiple_of(t * 16, 8), 16)], cntb)
    cnt = cntb[pl.ds(0, 16)][0]

    neg = jnp.full((16,), _NEG_INF, jnp.float32)

    def init_body(r, _):
        acc0[pl.ds(r * 16, 16)] = neg
        acc1[pl.ds(r * 16, 16)] = neg
        return 0

    lax.fori_loop(0, nt_pad * nh // 16, init_body, 0)

    nchunks = cnt >> 8

    def fetch(ci, b):
        boff = pl.multiple_of(t * rb + ci * GCH, 8)
        pltpu.sync_copy(csrc_ref.at[pl.ds(boff, GCH)], sidx[b])
        pltpu.sync_copy(cdst_ref.at[pl.ds(boff, GCH)],
                        dbuf[b].at[pl.ds(0, GCH)])
        pltpu.async_copy(b_ref.at[sidx[b]], rows[b], sem[b])

    @pl.when(nchunks > 0)
    def _():
        fetch(0, 0)

    def process(b):
        def edge_body(e2, _):
            e = e2 * 2
            a0 = (dbuf[b][pl.ds(e, 16)][0] - lo) * nh
            a1 = (dbuf[b][pl.ds(e + 1, 16)][0] - lo) * nh
            for j in range(nj):
                sl0 = pl.ds(a0 + j * 16, 16)
                sl1 = pl.ds(a1 + j * 16, 16)
                acc0[sl0] = jnp.maximum(acc0[sl0], rows[b][e, pl.ds(j * 16, 16)])
                acc1[sl1] = jnp.maximum(acc1[sl1],
                                        rows[b][e + 1, pl.ds(j * 16, 16)])
            return 0

        lax.fori_loop(0, GCH // 2, edge_body, 0)

    def pair_body(cp, _):
        for bb in (0, 1):
            ci = 2 * cp + bb

            @pl.when(ci < nchunks)
            def _(ci=ci, bb=bb):
                pltpu.make_async_copy(b_ref.at[sidx[bb]], rows[bb],
                                      sem[bb]).wait()

                @pl.when(ci + 1 < nchunks)
                def _():
                    fetch(ci + 1, 1 - bb)

                process(bb)
        return 0

    lax.fori_loop(0, (nchunks + 1) >> 1, pair_body, 0)

    def merge_body(r, _):
        sl = pl.ds(r * 16, 16)
        acc0[sl] = jnp.maximum(acc0[sl], acc1[sl])
        return 0

    lax.fori_loop(0, nt_pad * nh // 16, merge_body, 0)
    pltpu.sync_copy(acc0,
                    s_ref.at[pl.ds(pl.multiple_of(lo * nh, 8), nt_pad * nh)])


def _make_seg(nt_pad, nh, rb):
    mesh = plsc.VectorSubcoreMesh(core_axis_name="c", subcore_axis_name="s")
    return pl.kernel(
        functools.partial(_seg_body, nt_pad, nh, rb),
        out_type=[jax.ShapeDtypeStruct((NT * nt_pad * nh,), jnp.float32)],
        mesh=mesh,
        compiler_params=pltpu.CompilerParams(needs_layout_passes=False),
        scratch_types=[
            pltpu.VMEM((GCH,), jnp.int32),
            pltpu.VMEM((GCH,), jnp.int32),
            pltpu.VMEM((GCH + 16,), jnp.int32),
            pltpu.VMEM((GCH + 16,), jnp.int32),
            pltpu.VMEM((GCH, 128), jnp.float32),
            pltpu.VMEM((GCH, 128), jnp.float32),
            pltpu.VMEM((nt_pad * nh,), jnp.float32),
            pltpu.VMEM((nt_pad * nh,), jnp.float32),
            pltpu.VMEM((16,), jnp.int32),
            pltpu.SemaphoreType.DMA,
            pltpu.SemaphoreType.DMA,
        ],
    )


def _dot(a, b_t):
    return lax.dot_general(a, b_t, (((1,), (1,)), ((), ())),
                           preferred_element_type=jnp.float32,
                           precision=lax.Precision.HIGHEST)


def _first_tc_body(n_nodes, d_in, x_ref, w_ref, b_ref, bout_ref, cout_ref):
    x = x_ref[...]
    w = w_ref[...]
    wa = w[:, :d_in]
    wb = w[:, d_in:]
    bmat = _dot(x, wb)
    cmat = _dot(x, wa - wb) + b_ref[...]
    bout_ref[pl.ds(0, n_nodes), :] = jnp.concatenate(
        [bmat, jnp.zeros_like(bmat)], axis=1)
    bout_ref[pl.ds(n_nodes, 8), :] = jnp.full((8, 2 * bmat.shape[1]), _NEG_INF,
                                              jnp.float32)
    cout_ref[...] = cmat


def _mid_tc_body(n_nodes, nh, c_ref, s_ref, g_ref, be_ref, w_ref, b_ref,
                 bout_ref, cout_ref):
    agg = jnp.maximum(c_ref[...] + s_ref[pl.ds(0, n_nodes), :], 0.0)
    mean = jnp.mean(agg, axis=0, keepdims=True)
    var = jnp.mean((agg - mean) ** 2, axis=0, keepdims=True)
    hn = (agg - mean) * lax.rsqrt(var + 1e-5) * g_ref[...] + be_ref[...]
    h = jnp.maximum(hn, 0.0)
    w = w_ref[...]
    wa = w[:, :nh]
    wb = w[:, nh:]
    bmat = _dot(h, wb)
    cmat = _dot(h, wa - wb) + b_ref[...]
    bout_ref[pl.ds(0, n_nodes), :] = jnp.concatenate(
        [bmat, jnp.zeros_like(bmat)], axis=1)
    bout_ref[pl.ds(n_nodes, 8), :] = jnp.full((8, 2 * bmat.shape[1]), _NEG_INF,
                                              jnp.float32)
    cout_ref[...] = cmat


def _final_tc_body(n_nodes, c_ref, s_ref, g_ref, be_ref, batch_ref, wl_ref,
                   bl_ref, out_ref):
    agg = jnp.maximum(c_ref[...] + s_ref[pl.ds(0, n_nodes), :], 0.0)
    mean = jnp.mean(agg, axis=0, keepdims=True)
    var = jnp.mean((agg - mean) ** 2, axis=0, keepdims=True)
    h = (agg - mean) * lax.rsqrt(var + 1e-5) * g_ref[...] + be_ref[...]
    gid = lax.broadcasted_iota(jnp.int32, (n_nodes, G), 1)
    oh = (batch_ref[...] == gid).astype(jnp.float32)
    sums = lax.dot_general(oh, h, (((0,), (0,)), ((), ())),
                           preferred_element_type=jnp.float32,
                           precision=lax.Precision.HIGHEST)
    counts = jnp.sum(oh, axis=0, keepdims=True)
    pooled = sums / jnp.maximum(counts.T, 1.0)
    out = _dot(pooled, wl_ref[...]) + bl_ref[...]
    m = jnp.max(out, axis=1, keepdims=True)
    e = jnp.exp(out - m)
    out_ref[...] = e / jnp.sum(e, axis=1, keepdims=True)


def kernel(x, edge_index, batch, W1, b1, g1, be1, W2, b2, g2, be2,
           W3, b3, g3, be3, Wl, bl):
    n, d = x.shape
    e = edge_index.shape[1]
    nh = W1.shape[0]
    nt_pad = ((n + NT - 1) // NT + 7) // 8 * 8

    prep = _make_prep(n, e, nt_pad)
    csrc, cdst, counts = prep(edge_index.reshape(-1))
    seg = _make_seg(nt_pad, nh, csrc.shape[0] // NT)

    b1r = b1.reshape(1, nh)
    b2r = b2.reshape(1, nh)
    b3r = b3.reshape(1, nh)
    g1r, be1r = g1.reshape(1, nh), be1.reshape(1, nh)
    g2r, be2r = g2.reshape(1, nh), be2.reshape(1, nh)
    g3r, be3r = g3.reshape(1, nh), be3.reshape(1, nh)
    blr = bl.reshape(1, -1)
    batch2 = batch.reshape(n, 1)

    first = pl.pallas_call(
        functools.partial(_first_tc_body, n, d),
        out_shape=[
            jax.ShapeDtypeStruct((n + 8, 2 * nh), jnp.float32),
            jax.ShapeDtypeStruct((n, nh), jnp.float32),
        ],
    )
    mid = pl.pallas_call(
        functools.partial(_mid_tc_body, n, nh),
        out_shape=[
            jax.ShapeDtypeStruct((n + 8, 2 * nh), jnp.float32),
            jax.ShapeDtypeStruct((n, nh), jnp.float32),
        ],
    )
    final = pl.pallas_call(
        functools.partial(_final_tc_body, n),
        out_shape=jax.ShapeDtypeStruct((G, Wl.shape[0]), jnp.float32),
    )

    n_pad = NT * nt_pad
    B1, C1 = first(x, W1, b1r)
    (S1,) = seg(csrc, cdst, counts, B1)
    B2, C2 = mid(C1, S1.reshape(n_pad, nh), g1r, be1r, W2, b2r)
    (S2,) = seg(csrc, cdst, counts, B2)
    B3, C3 = mid(C2, S2.reshape(n_pad, nh), g2r, be2r, W3, b3r)
    (S3,) = seg(csrc, cdst, counts, B3)
    return final(C3, S3.reshape(n_pad, nh), g3r, be3r, batch2, Wl, blr)


# speedup vs baseline: 4.9205x; 1.0544x over previous
import functools

import jax
import jax.numpy as jnp
from jax import lax
from jax.experimental import pallas as pl
from jax.experimental.pallas import tpu as pltpu
from jax.experimental.pallas import tpu_sc as plsc

G = 64
NC = 2
NS = 16
NT = NC * NS

ECH = 3200
CBUF = 1184
FLUSH = 1024
GCH = 128

_NEG_INF = float("-inf")


def _prep_body(nt_pad, n_edges, n_nodes, rb, edge_ref, csrc_ref, cdst_ref,
               cnt_ref, srcb0, dstb0, srcb1, dstb1, sb, db, cntb, esem0, esem1):
    c = lax.axis_index("c")
    s = lax.axis_index("s")
    t = s * NC + c
    lo = t * nt_pad
    hi = lo + nt_pad
    nvec2 = ECH // 32
    nch = n_edges // ECH
    srcb = (srcb0, srcb1)
    dstb = (dstb0, dstb1)
    esem = (esem0, esem1)

    def fetch(ci, b):
        off = pl.multiple_of(ci * ECH, 8)
        pltpu.async_copy(edge_ref.at[pl.ds(off, ECH)], srcb[b], esem[b])
        off2 = pl.multiple_of(n_edges + ci * ECH, 8)
        pltpu.async_copy(edge_ref.at[pl.ds(off2, ECH)], dstb[b], esem[b])

    def wait_fetch(ci, b):
        off = pl.multiple_of(ci * ECH, 8)
        pltpu.make_async_copy(edge_ref.at[pl.ds(off, ECH)], srcb[b],
                              esem[b]).wait()
        off2 = pl.multiple_of(n_edges + ci * ECH, 8)
        pltpu.make_async_copy(edge_ref.at[pl.ds(off2, ECH)], dstb[b],
                              esem[b]).wait()

    fetch(0, 0)

    def pair_body(cp, carry):
        for bb in (0, 1):
            ci = 2 * cp + bb
            wait_fetch(ci, bb)

            @pl.when(ci + 1 < nch)
            def _(ci=ci, bb=bb):
                fetch(ci + 1, 1 - bb)

            def vec_body(vi, carry2, bb=bb):
                woff, flushed = carry2
                sv0 = srcb[bb][pl.ds(vi * 32, 16)]
                dv0 = dstb[bb][pl.ds(vi * 32, 16)]
                sv1 = srcb[bb][pl.ds(vi * 32 + 16, 16)]
                dv1 = dstb[bb][pl.ds(vi * 32 + 16, 16)]
                m0 = (dv0 >= lo) & (dv0 < hi)
                m1 = (dv1 >= lo) & (dv1 < hi)
                c0 = plsc.cumsum(m0.astype(jnp.int32))
                c1 = plsc.cumsum(m1.astype(jnp.int32))
                n0 = c0[15]
                pos0 = woff + c0 - 1
                pos1 = woff + n0 + c1 - 1
                plsc.store_scatter(sb, [pos0], sv0, mask=m0)
                plsc.store_scatter(db, [pos0], dv0, mask=m0)
                plsc.store_scatter(sb, [pos1], sv1, mask=m1)
                plsc.store_scatter(db, [pos1], dv1, mask=m1)
                woff = woff + n0 + c1[15]
                do_flush = woff >= FLUSH

                @pl.when(do_flush)
                def _():
                    off = pl.multiple_of(t * rb + flushed, 8)
                    pltpu.sync_copy(sb.at[pl.ds(0, FLUSH)],
                                    csrc_ref.at[pl.ds(off, FLUSH)])
                    pltpu.sync_copy(db.at[pl.ds(0, FLUSH)],
                                    cdst_ref.at[pl.ds(off, FLUSH)])
                    sb[pl.ds(0, 16)] = sb[pl.ds(FLUSH, 16)]
                    sb[pl.ds(16, 16)] = sb[pl.ds(FLUSH + 16, 16)]
                    db[pl.ds(0, 16)] = db[pl.ds(FLUSH, 16)]
                    db[pl.ds(16, 16)] = db[pl.ds(FLUSH + 16, 16)]

                woff = jnp.where(do_flush, woff - FLUSH, woff)
                flushed = jnp.where(do_flush, flushed + FLUSH, flushed)
                return woff, flushed

            carry = lax.fori_loop(0, nvec2, vec_body, carry)
        return carry

    woff, flushed = lax.fori_loop(0, nch // 2, pair_body,
                                  (jnp.int32(0), jnp.int32(0)))

    for k in range(GCH // 16):
        sb[pl.ds(woff + k * 16, 16)] = jnp.full((16,), n_nodes, jnp.int32)
        db[pl.ds(woff + k * 16, 16)] = jnp.full((16,), 1, jnp.int32) * lo
    p8 = ((woff + GCH - 1) >> 7) << 7
    off = pl.multiple_of(t * rb + flushed, 8)
    pltpu.sync_copy(sb.at[pl.ds(0, CBUF - 32)],
                    csrc_ref.at[pl.ds(off, CBUF - 32)])
    pltpu.sync_copy(db.at[pl.ds(0, CBUF - 32)],
                    cdst_ref.at[pl.ds(off, CBUF - 32)])
    cntb[pl.ds(0, 16)] = jnp.full((16,), 1, jnp.int32) * (flushed + p8)
    pltpu.sync_copy(cntb, cnt_ref.at[pl.ds(pl.multiple_of(t * 16, 8), 16)])


def _make_prep(n_nodes, n_edges, nt_pad):
    rb = n_edges + CBUF + FLUSH
    rb = ((rb + 7) // 8) * 8
    mesh = plsc.VectorSubcoreMesh(core_axis_name="c", subcore_axis_name="s")
    return pl.kernel(
        functools.partial(_prep_body, nt_pad, n_edges, n_nodes, rb),
        out_type=[
            jax.ShapeDtypeStruct((NT * rb,), jnp.int32),
            jax.ShapeDtypeStruct((NT * rb,), jnp.int32),
            jax.ShapeDtypeStruct((NT * 16,), jnp.int32),
        ],
        mesh=mesh,
        compiler_params=pltpu.CompilerParams(needs_layout_passes=False),
        scratch_types=[
            pltpu.VMEM((ECH,), jnp.int32),
            pltpu.VMEM((ECH,), jnp.int32),
            pltpu.VMEM((ECH,), jnp.int32),
            pltpu.VMEM((ECH,), jnp.int32),
            pltpu.VMEM((CBUF,), jnp.int32),
            pltpu.VMEM((CBUF,), jnp.int32),
            pltpu.VMEM((16,), jnp.int32),
            pltpu.SemaphoreType.DMA,
            pltpu.SemaphoreType.DMA,
        ],
    )


def _seg_body(nt_pad, nh, rb, csrc_ref, cdst_ref, cnt_ref, b_ref, s_ref,
              sidx0, sidx1, dbuf0, dbuf1, rows0, rows1, acc0, acc1, acc2,
              acc3, cntb, sem0, sem1):
    c = lax.axis_index("c")
    s = lax.axis_index("s")
    t = s * NC + c
    lo = t * nt_pad
    nj = nh // 16
    sidx = (sidx0, sidx1)
    dbuf = (dbuf0, dbuf1)
    rows = (rows0, rows1)
    sem = (sem0, sem1)

    pltpu.sync_copy(cnt_ref.at[pl.ds(pl.multiple_of(t * 16, 8), 16)], cntb)
    cnt = cntb[pl.ds(0, 16)][0]

    neg = jnp.full((16,), _NEG_INF, jnp.float32)

    accs = (acc0, acc1, acc2, acc3)

    def init_body(r, _):
        for a in accs:
            a[pl.ds(r * 16, 16)] = neg
        return 0

    lax.fori_loop(0, nt_pad * nh // 16, init_body, 0)

    nchunks = cnt >> 7

    def fetch(ci, b):
        boff = pl.multiple_of(t * rb + ci * GCH, 8)
        pltpu.sync_copy(csrc_ref.at[pl.ds(boff, GCH)], sidx[b])
        pltpu.sync_copy(cdst_ref.at[pl.ds(boff, GCH)],
                        dbuf[b].at[pl.ds(0, GCH)])
        pltpu.async_copy(b_ref.at[sidx[b]], rows[b], sem[b])

    @pl.when(nchunks > 0)
    def _():
        fetch(0, 0)

    def process(b):
        def edge_body(e4, _):
            e = e4 * 4
            addr = [(dbuf[b][pl.ds(e + i, 16)][0] - lo) * nh for i in range(4)]
            for j in range(nj):
                for i in range(4):
                    sl = pl.ds(addr[i] + j * 16, 16)
                    accs[i][sl] = jnp.maximum(
                        accs[i][sl], rows[b][e + i, pl.ds(j * 16, 16)])
            return 0

        lax.fori_loop(0, GCH // 4, edge_body, 0)

    def pair_body(cp, _):
        for bb in (0, 1):
            ci = 2 * cp + bb

            @pl.when(ci < nchunks)
            def _(ci=ci, bb=bb):
                pltpu.make_async_copy(b_ref.at[sidx[bb]], rows[bb],
                                      sem[bb]).wait()

                @pl.when(ci + 1 < nchunks)
                def _():
                    fetch(ci + 1, 1 - bb)

                process(bb)
        return 0

    lax.fori_loop(0, (nchunks + 1) >> 1, pair_body, 0)

    def merge_body(r, _):
        sl = pl.ds(r * 16, 16)
        acc0[sl] = jnp.maximum(jnp.maximum(acc0[sl], acc1[sl]),
                               jnp.maximum(acc2[sl], acc3[sl]))
        return 0

    lax.fori_loop(0, nt_pad * nh // 16, merge_body, 0)
    pltpu.sync_copy(acc0,
                    s_ref.at[pl.ds(pl.multiple_of(lo * nh, 8), nt_pad * nh)])


def _make_seg(nt_pad, nh, rb):
    mesh = plsc.VectorSubcoreMesh(core_axis_name="c", subcore_axis_name="s")
    return pl.kernel(
        functools.partial(_seg_body, nt_pad, nh, rb),
        out_type=[jax.ShapeDtypeStruct((NT * nt_pad * nh,), jnp.float32)],
        mesh=mesh,
        compiler_params=pltpu.CompilerParams(needs_layout_passes=False),
        scratch_types=[
            pltpu.VMEM((GCH,), jnp.int32),
            pltpu.VMEM((GCH,), jnp.int32),
            pltpu.VMEM((GCH + 16,), jnp.int32),
            pltpu.VMEM((GCH + 16,), jnp.int32),
            pltpu.VMEM((GCH, 128), jnp.float32),
            pltpu.VMEM((GCH, 128), jnp.float32),
            pltpu.VMEM((nt_pad * nh,), jnp.float32),
            pltpu.VMEM((nt_pad * nh,), jnp.float32),
            pltpu.VMEM((nt_pad * nh,), jnp.float32),
            pltpu.VMEM((nt_pad * nh,), jnp.float32),
            pltpu.VMEM((16,), jnp.int32),
            pltpu.SemaphoreType.DMA,
            pltpu.SemaphoreType.DMA,
        ],
    )


def _dot(a, b_t):
    return lax.dot_general(a, b_t, (((1,), (1,)), ((), ())),
                           preferred_element_type=jnp.float32,
                           precision=lax.Precision.HIGHEST)


def _first_tc_body(n_nodes, d_in, x_ref, w_ref, b_ref, bout_ref, cout_ref):
    x = x_ref[...]
    w = w_ref[...]
    wa = w[:, :d_in]
    wb = w[:, d_in:]
    bmat = _dot(x, wb)
    cmat = _dot(x, wa - wb) + b_ref[...]
    bout_ref[pl.ds(0, n_nodes), :] = jnp.concatenate(
        [bmat, jnp.zeros_like(bmat)], axis=1)
    bout_ref[pl.ds(n_nodes, 8), :] = jnp.full((8, 2 * bmat.shape[1]), _NEG_INF,
                                              jnp.float32)
    cout_ref[...] = cmat


def _mid_tc_body(n_nodes, nh, c_ref, s_ref, g_ref, be_ref, w_ref, b_ref,
                 bout_ref, cout_ref):
    agg = jnp.maximum(c_ref[...] + s_ref[pl.ds(0, n_nodes), :], 0.0)
    mean = jnp.mean(agg, axis=0, keepdims=True)
    var = jnp.mean((agg - mean) ** 2, axis=0, keepdims=True)
    hn = (agg - mean) * lax.rsqrt(var + 1e-5) * g_ref[...] + be_ref[...]
    h = jnp.maximum(hn, 0.0)
    w = w_ref[...]
    wa = w[:, :nh]
    wb = w[:, nh:]
    bmat = _dot(h, wb)
    cmat = _dot(h, wa - wb) + b_ref[...]
    bout_ref[pl.ds(0, n_nodes), :] = jnp.concatenate(
        [bmat, jnp.zeros_like(bmat)], axis=1)
    bout_ref[pl.ds(n_nodes, 8), :] = jnp.full((8, 2 * bmat.shape[1]), _NEG_INF,
                                              jnp.float32)
    cout_ref[...] = cmat


def _final_tc_body(n_nodes, c_ref, s_ref, g_ref, be_ref, batch_ref, wl_ref,
                   bl_ref, out_ref):
    agg = jnp.maximum(c_ref[...] + s_ref[pl.ds(0, n_nodes), :], 0.0)
    mean = jnp.mean(agg, axis=0, keepdims=True)
    var = jnp.mean((agg - mean) ** 2, axis=0, keepdims=True)
    h = (agg - mean) * lax.rsqrt(var + 1e-5) * g_ref[...] + be_ref[...]
    gid = lax.broadcasted_iota(jnp.int32, (n_nodes, G), 1)
    oh = (batch_ref[...] == gid).astype(jnp.float32)
    sums = lax.dot_general(oh, h, (((0,), (0,)), ((), ())),
                           preferred_element_type=jnp.float32,
                           precision=lax.Precision.HIGHEST)
    counts = jnp.sum(oh, axis=0, keepdims=True)
    pooled = sums / jnp.maximum(counts.T, 1.0)
    out = _dot(pooled, wl_ref[...]) + bl_ref[...]
    m = jnp.max(out, axis=1, keepdims=True)
    e = jnp.exp(out - m)
    out_ref[...] = e / jnp.sum(e, axis=1, keepdims=True)


def kernel(x, edge_index, batch, W1, b1, g1, be1, W2, b2, g2, be2,
           W3, b3, g3, be3, Wl, bl):
    n, d = x.shape
    e = edge_index.shape[1]
    nh = W1.shape[0]
    nt_pad = ((n + NT - 1) // NT + 7) // 8 * 8

    prep = _make_prep(n, e, nt_pad)
    csrc, cdst, counts = prep(edge_index.reshape(-1))
    seg = _make_seg(nt_pad, nh, csrc.shape[0] // NT)

    b1r = b1.reshape(1, nh)
    b2r = b2.reshape(1, nh)
    b3r = b3.reshape(1, nh)
    g1r, be1r = g1.reshape(1, nh), be1.reshape(1, nh)
    g2r, be2r = g2.reshape(1, nh), be2.reshape(1, nh)
    g3r, be3r = g3.reshape(1, nh), be3.reshape(1, nh)
    blr = bl.reshape(1, -1)
    batch2 = batch.reshape(n, 1)

    first = pl.pallas_call(
        functools.partial(_first_tc_body, n, d),
        out_shape=[
            jax.ShapeDtypeStruct((n + 8, 2 * nh), jnp.float32),
            jax.ShapeDtypeStruct((n, nh), jnp.float32),
        ],
    )
    mid = pl.pallas_call(
        functools.partial(_mid_tc_body, n, nh),
        out_shape=[
            jax.ShapeDtypeStruct((n + 8, 2 * nh), jnp.float32),
            jax.ShapeDtypeStruct((n, nh), jnp.float32),
        ],
    )
    final = pl.pallas_call(
        functools.partial(_final_tc_body, n),
        out_shape=jax.ShapeDtypeStruct((G, Wl.shape[0]), jnp.float32),
    )

    n_pad = NT * nt_pad
    B1, C1 = first(x, W1, b1r)
    (S1,) = seg(csrc, cdst, counts, B1)
    B2, C2 = mid(C1, S1.reshape(n_pad, nh), g1r, be1r, W2, b2r)
    (S2,) = seg(csrc, cdst, counts, B2)
    B3, C3 = mid(C2, S2.reshape(n_pad, nh), g2r, be2r, W3, b3r)
    (S3,) = seg(csrc, cdst, counts, B3)
    return final(C3, S3.reshape(n_pad, nh), g3r, be3r, batch2, Wl, blr)


# speedup vs baseline: 5.1884x; 1.0544x over previous
import functools

import jax
import jax.numpy as jnp
from jax import lax
from jax.experimental import pallas as pl
from jax.experimental.pallas import tpu as pltpu
from jax.experimental.pallas import tpu_sc as plsc

G = 64
NC = 2
NS = 16
NT = NC * NS

ECH = 3200
CBUF = 1184
FLUSH = 1024
GCH = 128

_NEG_INF = float("-inf")
_NH_STATIC = [64]


def _prep_body(nt_pad, n_edges, n_nodes, rb, edge_ref, csrc_ref, cdst_ref,
               cnt_ref, srcb0, dstb0, srcb1, dstb1, sb, db, cntb, esem0, esem1):
    c = lax.axis_index("c")
    s = lax.axis_index("s")
    t = s * NC + c
    lo = t * nt_pad
    hi = lo + nt_pad
    nvec2 = ECH // 32
    nch = n_edges // ECH
    srcb = (srcb0, srcb1)
    dstb = (dstb0, dstb1)
    esem = (esem0, esem1)

    def fetch(ci, b):
        off = pl.multiple_of(ci * ECH, 8)
        pltpu.async_copy(edge_ref.at[pl.ds(off, ECH)], srcb[b], esem[b])
        off2 = pl.multiple_of(n_edges + ci * ECH, 8)
        pltpu.async_copy(edge_ref.at[pl.ds(off2, ECH)], dstb[b], esem[b])

    def wait_fetch(ci, b):
        off = pl.multiple_of(ci * ECH, 8)
        pltpu.make_async_copy(edge_ref.at[pl.ds(off, ECH)], srcb[b],
                              esem[b]).wait()
        off2 = pl.multiple_of(n_edges + ci * ECH, 8)
        pltpu.make_async_copy(edge_ref.at[pl.ds(off2, ECH)], dstb[b],
                              esem[b]).wait()

    fetch(0, 0)

    def pair_body(cp, carry):
        for bb in (0, 1):
            ci = 2 * cp + bb
            wait_fetch(ci, bb)

            @pl.when(ci + 1 < nch)
            def _(ci=ci, bb=bb):
                fetch(ci + 1, 1 - bb)

            def vec_body(vi, carry2, bb=bb):
                woff, flushed = carry2
                sv0 = srcb[bb][pl.ds(vi * 32, 16)]
                dv0 = dstb[bb][pl.ds(vi * 32, 16)]
                sv1 = srcb[bb][pl.ds(vi * 32 + 16, 16)]
                dv1 = dstb[bb][pl.ds(vi * 32 + 16, 16)]
                m0 = (dv0 >= lo) & (dv0 < hi)
                m1 = (dv1 >= lo) & (dv1 < hi)
                c0 = plsc.cumsum(m0.astype(jnp.int32))
                c1 = plsc.cumsum(m1.astype(jnp.int32))
                n0 = c0[15]
                pos0 = woff + c0 - 1
                pos1 = woff + n0 + c1 - 1
                nhc = jnp.int32(_NH_STATIC[0])
                plsc.store_scatter(sb, [pos0], sv0, mask=m0)
                plsc.store_scatter(db, [pos0], (dv0 - lo) * nhc, mask=m0)
                plsc.store_scatter(sb, [pos1], sv1, mask=m1)
                plsc.store_scatter(db, [pos1], (dv1 - lo) * nhc, mask=m1)
                woff = woff + n0 + c1[15]
                do_flush = woff >= FLUSH

                @pl.when(do_flush)
                def _():
                    off = pl.multiple_of(t * rb + flushed, 8)
                    pltpu.sync_copy(sb.at[pl.ds(0, FLUSH)],
                                    csrc_ref.at[pl.ds(off, FLUSH)])
                    pltpu.sync_copy(db.at[pl.ds(0, FLUSH)],
                                    cdst_ref.at[pl.ds(off, FLUSH)])
                    sb[pl.ds(0, 16)] = sb[pl.ds(FLUSH, 16)]
                    sb[pl.ds(16, 16)] = sb[pl.ds(FLUSH + 16, 16)]
                    db[pl.ds(0, 16)] = db[pl.ds(FLUSH, 16)]
                    db[pl.ds(16, 16)] = db[pl.ds(FLUSH + 16, 16)]

                woff = jnp.where(do_flush, woff - FLUSH, woff)
                flushed = jnp.where(do_flush, flushed + FLUSH, flushed)
                return woff, flushed

            carry = lax.fori_loop(0, nvec2, vec_body, carry)
        return carry

    woff, flushed = lax.fori_loop(0, nch // 2, pair_body,
                                  (jnp.int32(0), jnp.int32(0)))

    for k in range(GCH // 16):
        sb[pl.ds(woff + k * 16, 16)] = jnp.full((16,), n_nodes, jnp.int32)
        db[pl.ds(woff + k * 16, 16)] = jnp.zeros((16,), jnp.int32)
    p8 = ((woff + GCH - 1) >> 7) << 7
    off = pl.multiple_of(t * rb + flushed, 8)
    pltpu.sync_copy(sb.at[pl.ds(0, CBUF - 32)],
                    csrc_ref.at[pl.ds(off, CBUF - 32)])
    pltpu.sync_copy(db.at[pl.ds(0, CBUF - 32)],
                    cdst_ref.at[pl.ds(off, CBUF - 32)])
    cntb[pl.ds(0, 16)] = jnp.full((16,), 1, jnp.int32) * (flushed + p8)
    pltpu.sync_copy(cntb, cnt_ref.at[pl.ds(pl.multiple_of(t * 16, 8), 16)])


def _make_prep(n_nodes, n_edges, nt_pad, nh):
    _NH_STATIC[0] = nh
    rb = n_edges + CBUF + FLUSH
    rb = ((rb + 7) // 8) * 8
    mesh = plsc.VectorSubcoreMesh(core_axis_name="c", subcore_axis_name="s")
    return pl.kernel(
        functools.partial(_prep_body, nt_pad, n_edges, n_nodes, rb),
        out_type=[
            jax.ShapeDtypeStruct((NT * rb,), jnp.int32),
            jax.ShapeDtypeStruct((NT * rb,), jnp.int32),
            jax.ShapeDtypeStruct((NT * 16,), jnp.int32),
        ],
        mesh=mesh,
        compiler_params=pltpu.CompilerParams(needs_layout_passes=False),
        scratch_types=[
            pltpu.VMEM((ECH,), jnp.int32),
            pltpu.VMEM((ECH,), jnp.int32),
            pltpu.VMEM((ECH,), jnp.int32),
            pltpu.VMEM((ECH,), jnp.int32),
            pltpu.VMEM((CBUF,), jnp.int32),
            pltpu.VMEM((CBUF,), jnp.int32),
            pltpu.VMEM((16,), jnp.int32),
            pltpu.SemaphoreType.DMA,
            pltpu.SemaphoreType.DMA,
        ],
    )


def _seg_body(nt_pad, nh, rb, csrc_ref, cdst_ref, cnt_ref, b_ref, s_ref,
              sidx0, sidx1, dbuf0, dbuf1, rows0, rows1, acc0, acc1, acc2,
              acc3, cntb, sem0, sem1):
    c = lax.axis_index("c")
    s = lax.axis_index("s")
    t = s * NC + c
    lo = t * nt_pad
    nj = nh // 16
    sidx = (sidx0, sidx1)
    dbuf = (dbuf0, dbuf1)
    rows = (rows0, rows1)
    sem = (sem0, sem1)

    pltpu.sync_copy(cnt_ref.at[pl.ds(pl.multiple_of(t * 16, 8), 16)], cntb)
    cnt = cntb[pl.ds(0, 16)][0]

    neg = jnp.full((16,), _NEG_INF, jnp.float32)

    accs = (acc0, acc1, acc2, acc3)

    def init_body(r, _):
        for a in accs:
            a[pl.ds(r * 16, 16)] = neg
        return 0

    lax.fori_loop(0, nt_pad * nh // 16, init_body, 0)

    nchunks = cnt >> 7

    def fetch(ci, b):
        boff = pl.multiple_of(t * rb + ci * GCH, 8)
        pltpu.sync_copy(csrc_ref.at[pl.ds(boff, GCH)], sidx[b])
        pltpu.sync_copy(cdst_ref.at[pl.ds(boff, GCH)],
                        dbuf[b].at[pl.ds(0, GCH)])
        pltpu.async_copy(b_ref.at[sidx[b]], rows[b], sem[b])

    @pl.when(nchunks > 0)
    def _():
        fetch(0, 0)

    def process(b):
        def edge_body(e16, _):
            e = e16 * 16
            dvec = dbuf[b][pl.ds(e, 16)]
            for i4 in range(4):
                addr = [dvec[i4 * 4 + i] for i in range(4)]
                for j in range(nj):
                    for i in range(4):
                        sl = pl.ds(addr[i] + j * 16, 16)
                        accs[i][sl] = jnp.maximum(
                            accs[i][sl],
                            rows[b][e + i4 * 4 + i, pl.ds(j * 16, 16)])
            return 0

        lax.fori_loop(0, GCH // 16, edge_body, 0)

    def pair_body(cp, _):
        for bb in (0, 1):
            ci = 2 * cp + bb

            @pl.when(ci < nchunks)
            def _(ci=ci, bb=bb):
                pltpu.make_async_copy(b_ref.at[sidx[bb]], rows[bb],
                                      sem[bb]).wait()

                @pl.when(ci + 1 < nchunks)
                def _():
                    fetch(ci + 1, 1 - bb)

                process(bb)
        return 0

    lax.fori_loop(0, (nchunks + 1) >> 1, pair_body, 0)

    def merge_body(r, _):
        sl = pl.ds(r * 16, 16)
        acc0[sl] = jnp.maximum(jnp.maximum(acc0[sl], acc1[sl]),
                               jnp.maximum(acc2[sl], acc3[sl]))
        return 0

    lax.fori_loop(0, nt_pad * nh // 16, merge_body, 0)
    pltpu.sync_copy(acc0,
                    s_ref.at[pl.ds(pl.multiple_of(lo * nh, 8), nt_pad * nh)])


def _make_seg(nt_pad, nh, rb):
    mesh = plsc.VectorSubcoreMesh(core_axis_name="c", subcore_axis_name="s")
    return pl.kernel(
        functools.partial(_seg_body, nt_pad, nh, rb),
        out_type=[jax.ShapeDtypeStruct((NT * nt_pad * nh,), jnp.float32)],
        mesh=mesh,
        compiler_params=pltpu.CompilerParams(needs_layout_passes=False),
        scratch_types=[
            pltpu.VMEM((GCH,), jnp.int32),
            pltpu.VMEM((GCH,), jnp.int32),
            pltpu.VMEM((GCH + 16,), jnp.int32),
            pltpu.VMEM((GCH + 16,), jnp.int32),
            pltpu.VMEM((GCH, 128), jnp.float32),
            pltpu.VMEM((GCH, 128), jnp.float32),
            pltpu.VMEM((nt_pad * nh,), jnp.float32),
            pltpu.VMEM((nt_pad * nh,), jnp.float32),
            pltpu.VMEM((nt_pad * nh,), jnp.float32),
            pltpu.VMEM((nt_pad * nh,), jnp.float32),
            pltpu.VMEM((16,), jnp.int32),
            pltpu.SemaphoreType.DMA,
            pltpu.SemaphoreType.DMA,
        ],
    )


def _dot(a, b_t):
    return lax.dot_general(a, b_t, (((1,), (1,)), ((), ())),
                           preferred_element_type=jnp.float32,
                           precision=lax.Precision.HIGHEST)


def _first_tc_body(n_nodes, d_in, x_ref, w_ref, b_ref, bout_ref, cout_ref):
    x = x_ref[...]
    w = w_ref[...]
    wa = w[:, :d_in]
    wb = w[:, d_in:]
    bmat = _dot(x, wb)
    cmat = _dot(x, wa - wb) + b_ref[...]
    bout_ref[pl.ds(0, n_nodes), :] = jnp.concatenate(
        [bmat, jnp.zeros_like(bmat)], axis=1)
    bout_ref[pl.ds(n_nodes, 8), :] = jnp.full((8, 2 * bmat.shape[1]), _NEG_INF,
                                              jnp.float32)
    cout_ref[...] = cmat


def _mid_tc_body(n_nodes, nh, c_ref, s_ref, g_ref, be_ref, w_ref, b_ref,
                 bout_ref, cout_ref):
    agg = jnp.maximum(c_ref[...] + s_ref[pl.ds(0, n_nodes), :], 0.0)
    mean = jnp.mean(agg, axis=0, keepdims=True)
    var = jnp.mean((agg - mean) ** 2, axis=0, keepdims=True)
    hn = (agg - mean) * lax.rsqrt(var + 1e-5) * g_ref[...] + be_ref[...]
    h = jnp.maximum(hn, 0.0)
    w = w_ref[...]
    wa = w[:, :nh]
    wb = w[:, nh:]
    bmat = _dot(h, wb)
    cmat = _dot(h, wa - wb) + b_ref[...]
    bout_ref[pl.ds(0, n_nodes), :] = jnp.concatenate(
        [bmat, jnp.zeros_like(bmat)], axis=1)
    bout_ref[pl.ds(n_nodes, 8), :] = jnp.full((8, 2 * bmat.shape[1]), _NEG_INF,
                                              jnp.float32)
    cout_ref[...] = cmat


def _final_tc_body(n_nodes, c_ref, s_ref, g_ref, be_ref, batch_ref, wl_ref,
                   bl_ref, out_ref):
    agg = jnp.maximum(c_ref[...] + s_ref[pl.ds(0, n_nodes), :], 0.0)
    mean = jnp.mean(agg, axis=0, keepdims=True)
    var = jnp.mean((agg - mean) ** 2, axis=0, keepdims=True)
    h = (agg - mean) * lax.rsqrt(var + 1e-5) * g_ref[...] + be_ref[...]
    gid = lax.broadcasted_iota(jnp.int32, (n_nodes, G), 1)
    oh = (batch_ref[...] == gid).astype(jnp.float32)
    sums = lax.dot_general(oh, h, (((0,), (0,)), ((), ())),
                           preferred_element_type=jnp.float32,
                           precision=lax.Precision.HIGHEST)
    counts = jnp.sum(oh, axis=0, keepdims=True)
    pooled = sums / jnp.maximum(counts.T, 1.0)
    out = _dot(pooled, wl_ref[...]) + bl_ref[...]
    m = jnp.max(out, axis=1, keepdims=True)
    e = jnp.exp(out - m)
    out_ref[...] = e / jnp.sum(e, axis=1, keepdims=True)


def kernel(x, edge_index, batch, W1, b1, g1, be1, W2, b2, g2, be2,
           W3, b3, g3, be3, Wl, bl):
    n, d = x.shape
    e = edge_index.shape[1]
    nh = W1.shape[0]
    nt_pad = ((n + NT - 1) // NT + 7) // 8 * 8

    prep = _make_prep(n, e, nt_pad, nh)
    csrc, cdst, counts = prep(edge_index.reshape(-1))
    seg = _make_seg(nt_pad, nh, csrc.shape[0] // NT)

    b1r = b1.reshape(1, nh)
    b2r = b2.reshape(1, nh)
    b3r = b3.reshape(1, nh)
    g1r, be1r = g1.reshape(1, nh), be1.reshape(1, nh)
    g2r, be2r = g2.reshape(1, nh), be2.reshape(1, nh)
    g3r, be3r = g3.reshape(1, nh), be3.reshape(1, nh)
    blr = bl.reshape(1, -1)
    batch2 = batch.reshape(n, 1)

    first = pl.pallas_call(
        functools.partial(_first_tc_body, n, d),
        out_shape=[
            jax.ShapeDtypeStruct((n + 8, 2 * nh), jnp.float32),
            jax.ShapeDtypeStruct((n, nh), jnp.float32),
        ],
    )
    mid = pl.pallas_call(
        functools.partial(_mid_tc_body, n, nh),
        out_shape=[
            jax.ShapeDtypeStruct((n + 8, 2 * nh), jnp.float32),
            jax.ShapeDtypeStruct((n, nh), jnp.float32),
        ],
    )
    final = pl.pallas_call(
        functools.partial(_final_tc_body, n),
        out_shape=jax.ShapeDtypeStruct((G, Wl.shape[0]), jnp.float32),
    )

    n_pad = NT * nt_pad
    B1, C1 = first(x, W1, b1r)
    (S1,) = seg(csrc, cdst, counts, B1)
    B2, C2 = mid(C1, S1.reshape(n_pad, nh), g1r, be1r, W2, b2r)
    (S2,) = seg(csrc, cdst, counts, B2)
    B3, C3 = mid(C2, S2.reshape(n_pad, nh), g2r, be2r, W3, b3r)
    (S3,) = seg(csrc, cdst, counts, B3)
    return final(C3, S3.reshape(n_pad, nh), g3r, be3r, batch2, Wl, blr)


# speedup vs baseline: 5.7403x; 1.1064x over previous
import functools

import jax
import jax.numpy as jnp
from jax import lax
from jax.experimental import pallas as pl
from jax.experimental.pallas import tpu as pltpu
from jax.experimental.pallas import tpu_sc as plsc

G = 64
NC = 2
NS = 16
NT = NC * NS

ECH = 3200
CBUF = 1216
FLUSH = 1024
GCH = 128

_NEG_INF = float("-inf")
_NH_STATIC = [64]


def _prep_body(nt_pad, n_edges, n_nodes, rb, edge_ref, csrc_ref, cdst_ref,
               cnt_ref, srcb0, dstb0, srcb1, dstb1, sb, db, cntb, esem0, esem1):
    c = lax.axis_index("c")
    s = lax.axis_index("s")
    t = s * NC + c
    lo = t * nt_pad
    hi = lo + nt_pad
    nvec2 = ECH // 32
    nch = n_edges // ECH
    srcb = (srcb0, srcb1)
    dstb = (dstb0, dstb1)
    esem = (esem0, esem1)

    def fetch(ci, b):
        off = pl.multiple_of(ci * ECH, 8)
        pltpu.async_copy(edge_ref.at[pl.ds(off, ECH)], srcb[b], esem[b])
        off2 = pl.multiple_of(n_edges + ci * ECH, 8)
        pltpu.async_copy(edge_ref.at[pl.ds(off2, ECH)], dstb[b], esem[b])

    def wait_fetch(ci, b):
        off = pl.multiple_of(ci * ECH, 8)
        pltpu.make_async_copy(edge_ref.at[pl.ds(off, ECH)], srcb[b],
                              esem[b]).wait()
        off2 = pl.multiple_of(n_edges + ci * ECH, 8)
        pltpu.make_async_copy(edge_ref.at[pl.ds(off2, ECH)], dstb[b],
                              esem[b]).wait()

    fetch(0, 0)

    def pair_body(cp, carry):
        for bb in (0, 1):
            ci = 2 * cp + bb
            wait_fetch(ci, bb)

            @pl.when(ci + 1 < nch)
            def _(ci=ci, bb=bb):
                fetch(ci + 1, 1 - bb)

            def vec_body(vi, carry2, bb=bb):
                woff, flushed = carry2
                nhc = jnp.int32(_NH_STATIC[0])
                svs, dvs, ms, cs = [], [], [], []
                for u in range(4):
                    sv = srcb[bb][pl.ds(vi * 64 + u * 16, 16)]
                    dv = dstb[bb][pl.ds(vi * 64 + u * 16, 16)]
                    m = (dv >= lo) & (dv < hi)
                    svs.append(sv)
                    dvs.append(dv)
                    ms.append(m)
                    cs.append(plsc.cumsum(m.astype(jnp.int32)))
                base = woff
                for u in range(4):
                    pos = base + cs[u] - 1
                    plsc.store_scatter(sb, [pos], svs[u], mask=ms[u])
                    plsc.store_scatter(db, [pos], (dvs[u] - lo) * nhc,
                                       mask=ms[u])
                    base = base + cs[u][15]
                woff = base
                do_flush = woff >= FLUSH

                @pl.when(do_flush)
                def _():
                    off = pl.multiple_of(t * rb + flushed, 8)
                    pltpu.sync_copy(sb.at[pl.ds(0, FLUSH)],
                                    csrc_ref.at[pl.ds(off, FLUSH)])
                    pltpu.sync_copy(db.at[pl.ds(0, FLUSH)],
                                    cdst_ref.at[pl.ds(off, FLUSH)])
                    for u in range(4):
                        sb[pl.ds(u * 16, 16)] = sb[pl.ds(FLUSH + u * 16, 16)]
                        db[pl.ds(u * 16, 16)] = db[pl.ds(FLUSH + u * 16, 16)]

                woff = jnp.where(do_flush, woff - FLUSH, woff)
                flushed = jnp.where(do_flush, flushed + FLUSH, flushed)
                return woff, flushed

            carry = lax.fori_loop(0, nvec2 // 2, vec_body, carry)
        return carry

    woff, flushed = lax.fori_loop(0, nch // 2, pair_body,
                                  (jnp.int32(0), jnp.int32(0)))

    for k in range(GCH // 16):
        sb[pl.ds(woff + k * 16, 16)] = jnp.full((16,), n_nodes, jnp.int32)
        db[pl.ds(woff + k * 16, 16)] = jnp.zeros((16,), jnp.int32)
    p8 = ((woff + GCH - 1) >> 7) << 7
    off = pl.multiple_of(t * rb + flushed, 8)
    pltpu.sync_copy(sb.at[pl.ds(0, CBUF - 64)],
                    csrc_ref.at[pl.ds(off, CBUF - 64)])
    pltpu.sync_copy(db.at[pl.ds(0, CBUF - 64)],
                    cdst_ref.at[pl.ds(off, CBUF - 64)])
    cntb[pl.ds(0, 16)] = jnp.full((16,), 1, jnp.int32) * (flushed + p8)
    pltpu.sync_copy(cntb, cnt_ref.at[pl.ds(pl.multiple_of(t * 16, 8), 16)])


def _make_prep(n_nodes, n_edges, nt_pad, nh):
    _NH_STATIC[0] = nh
    rb = n_edges + CBUF + FLUSH
    rb = ((rb + 7) // 8) * 8
    mesh = plsc.VectorSubcoreMesh(core_axis_name="c", subcore_axis_name="s")
    return pl.kernel(
        functools.partial(_prep_body, nt_pad, n_edges, n_nodes, rb),
        out_type=[
            jax.ShapeDtypeStruct((NT * rb,), jnp.int32),
            jax.ShapeDtypeStruct((NT * rb,), jnp.int32),
            jax.ShapeDtypeStruct((NT * 16,), jnp.int32),
        ],
        mesh=mesh,
        compiler_params=pltpu.CompilerParams(needs_layout_passes=False),
        scratch_types=[
            pltpu.VMEM((ECH,), jnp.int32),
            pltpu.VMEM((ECH,), jnp.int32),
            pltpu.VMEM((ECH,), jnp.int32),
            pltpu.VMEM((ECH,), jnp.int32),
            pltpu.VMEM((CBUF,), jnp.int32),
            pltpu.VMEM((CBUF,), jnp.int32),
            pltpu.VMEM((16,), jnp.int32),
            pltpu.SemaphoreType.DMA,
            pltpu.SemaphoreType.DMA,
        ],
    )


def _seg_body(nt_pad, nh, rb, csrc_ref, cdst_ref, cnt_ref, b_ref, s_ref,
              sidx0, sidx1, dbuf0, dbuf1, rows0, rows1, acc0, acc1, acc2,
              acc3, cntb, sem0, sem1):
    c = lax.axis_index("c")
    s = lax.axis_index("s")
    t = s * NC + c
    lo = t * nt_pad
    nj = nh // 16
    sidx = (sidx0, sidx1)
    dbuf = (dbuf0, dbuf1)
    rows = (rows0, rows1)
    sem = (sem0, sem1)

    pltpu.sync_copy(cnt_ref.at[pl.ds(pl.multiple_of(t * 16, 8), 16)], cntb)
    cnt = cntb[pl.ds(0, 16)][0]

    neg = jnp.full((16,), _NEG_INF, jnp.float32)

    accs = (acc0, acc1, acc2, acc3)

    def init_body(r, _):
        for a in accs:
            a[pl.ds(r * 16, 16)] = neg
        return 0

    lax.fori_loop(0, nt_pad * nh // 16, init_body, 0)

    nchunks = cnt >> 7

    def fetch(ci, b):
        boff = pl.multiple_of(t * rb + ci * GCH, 8)
        pltpu.sync_copy(csrc_ref.at[pl.ds(boff, GCH)], sidx[b])
        pltpu.sync_copy(cdst_ref.at[pl.ds(boff, GCH)],
                        dbuf[b].at[pl.ds(0, GCH)])
        pltpu.async_copy(b_ref.at[sidx[b]], rows[b], sem[b])

    @pl.when(nchunks > 0)
    def _():
        fetch(0, 0)

    def process(b):
        def edge_body(e16, _):
            e = e16 * 16
            dvec = dbuf[b][pl.ds(e, 16)]
            for i4 in range(4):
                addr = [dvec[i4 * 4 + i] for i in range(4)]
                for j in range(nj):
                    for i in range(4):
                        sl = pl.ds(addr[i] + j * 16, 16)
                        accs[i][sl] = jnp.maximum(
                            accs[i][sl],
                            rows[b][e + i4 * 4 + i, pl.ds(j * 16, 16)])
            return 0

        lax.fori_loop(0, GCH // 16, edge_body, 0)

    def pair_body(cp, _):
        for bb in (0, 1):
            ci = 2 * cp + bb

            @pl.when(ci < nchunks)
            def _(ci=ci, bb=bb):
                pltpu.make_async_copy(b_ref.at[sidx[bb]], rows[bb],
                                      sem[bb]).wait()

                @pl.when(ci + 1 < nchunks)
                def _():
                    fetch(ci + 1, 1 - bb)

                process(bb)
        return 0

    lax.fori_loop(0, (nchunks + 1) >> 1, pair_body, 0)

    def merge_body(r, _):
        sl = pl.ds(r * 16, 16)
        acc0[sl] = jnp.maximum(jnp.maximum(acc0[sl], acc1[sl]),
                               jnp.maximum(acc2[sl], acc3[sl]))
        return 0

    lax.fori_loop(0, nt_pad * nh // 16, merge_body, 0)
    pltpu.sync_copy(acc0,
                    s_ref.at[pl.ds(pl.multiple_of(lo * nh, 8), nt_pad * nh)])


def _make_seg(nt_pad, nh, rb):
    mesh = plsc.VectorSubcoreMesh(core_axis_name="c", subcore_axis_name="s")
    return pl.kernel(
        functools.partial(_seg_body, nt_pad, nh, rb),
        out_type=[jax.ShapeDtypeStruct((NT * nt_pad * nh,), jnp.float32)],
        mesh=mesh,
        compiler_params=pltpu.CompilerParams(needs_layout_passes=False),
        scratch_types=[
            pltpu.VMEM((GCH,), jnp.int32),
            pltpu.VMEM((GCH,), jnp.int32),
            pltpu.VMEM((GCH + 16,), jnp.int32),
            pltpu.VMEM((GCH + 16,), jnp.int32),
            pltpu.VMEM((GCH, 128), jnp.float32),
            pltpu.VMEM((GCH, 128), jnp.float32),
            pltpu.VMEM((nt_pad * nh,), jnp.float32),
            pltpu.VMEM((nt_pad * nh,), jnp.float32),
            pltpu.VMEM((nt_pad * nh,), jnp.float32),
            pltpu.VMEM((nt_pad * nh,), jnp.float32),
            pltpu.VMEM((16,), jnp.int32),
            pltpu.SemaphoreType.DMA,
            pltpu.SemaphoreType.DMA,
        ],
    )


def _dot(a, b_t):
    return lax.dot_general(a, b_t, (((1,), (1,)), ((), ())),
                           preferred_element_type=jnp.float32,
                           precision=lax.Precision.HIGHEST)


def _first_tc_body(n_nodes, d_in, x_ref, w_ref, b_ref, bout_ref, cout_ref):
    x = x_ref[...]
    w = w_ref[...]
    wa = w[:, :d_in]
    wb = w[:, d_in:]
    bmat = _dot(x, wb)
    cmat = _dot(x, wa - wb) + b_ref[...]
    bout_ref[pl.ds(0, n_nodes), :] = jnp.concatenate(
        [bmat, jnp.zeros_like(bmat)], axis=1)
    bout_ref[pl.ds(n_nodes, 8), :] = jnp.full((8, 2 * bmat.shape[1]), _NEG_INF,
                                              jnp.float32)
    cout_ref[...] = cmat


def _mid_tc_body(n_nodes, nh, c_ref, s_ref, g_ref, be_ref, w_ref, b_ref,
                 bout_ref, cout_ref):
    agg = jnp.maximum(c_ref[...] + s_ref[pl.ds(0, n_nodes), :], 0.0)
    mean = jnp.mean(agg, axis=0, keepdims=True)
    var = jnp.mean((agg - mean) ** 2, axis=0, keepdims=True)
    hn = (agg - mean) * lax.rsqrt(var + 1e-5) * g_ref[...] + be_ref[...]
    h = jnp.maximum(hn, 0.0)
    w = w_ref[...]
    wa = w[:, :nh]
    wb = w[:, nh:]
    bmat = _dot(h, wb)
    cmat = _dot(h, wa - wb) + b_ref[...]
    bout_ref[pl.ds(0, n_nodes), :] = jnp.concatenate(
        [bmat, jnp.zeros_like(bmat)], axis=1)
    bout_ref[pl.ds(n_nodes, 8), :] = jnp.full((8, 2 * bmat.shape[1]), _NEG_INF,
                                              jnp.float32)
    cout_ref[...] = cmat


def _final_tc_body(n_nodes, c_ref, s_ref, g_ref, be_ref, batch_ref, wl_ref,
                   bl_ref, out_ref):
    agg = jnp.maximum(c_ref[...] + s_ref[pl.ds(0, n_nodes), :], 0.0)
    mean = jnp.mean(agg, axis=0, keepdims=True)
    var = jnp.mean((agg - mean) ** 2, axis=0, keepdims=True)
    h = (agg - mean) * lax.rsqrt(var + 1e-5) * g_ref[...] + be_ref[...]
    gid = lax.broadcasted_iota(jnp.int32, (n_nodes, G), 1)
    oh = (batch_ref[...] == gid).astype(jnp.float32)
    sums = lax.dot_general(oh, h, (((0,), (0,)), ((), ())),
                           preferred_element_type=jnp.float32,
                           precision=lax.Precision.HIGHEST)
    counts = jnp.sum(oh, axis=0, keepdims=True)
    pooled = sums / jnp.maximum(counts.T, 1.0)
    out = _dot(pooled, wl_ref[...]) + bl_ref[...]
    m = jnp.max(out, axis=1, keepdims=True)
    e = jnp.exp(out - m)
    out_ref[...] = e / jnp.sum(e, axis=1, keepdims=True)


def kernel(x, edge_index, batch, W1, b1, g1, be1, W2, b2, g2, be2,
           W3, b3, g3, be3, Wl, bl):
    n, d = x.shape
    e = edge_index.shape[1]
    nh = W1.shape[0]
    nt_pad = ((n + NT - 1) // NT + 7) // 8 * 8

    prep = _make_prep(n, e, nt_pad, nh)
    csrc, cdst, counts = prep(edge_index.reshape(-1))
    seg = _make_seg(nt_pad, nh, csrc.shape[0] // NT)

    b1r = b1.reshape(1, nh)
    b2r = b2.reshape(1, nh)
    b3r = b3.reshape(1, nh)
    g1r, be1r = g1.reshape(1, nh), be1.reshape(1, nh)
    g2r, be2r = g2.reshape(1, nh), be2.reshape(1, nh)
    g3r, be3r = g3.reshape(1, nh), be3.reshape(1, nh)
    blr = bl.reshape(1, -1)
    batch2 = batch.reshape(n, 1)

    first = pl.pallas_call(
        functools.partial(_first_tc_body, n, d),
        out_shape=[
            jax.ShapeDtypeStruct((n + 8, 2 * nh), jnp.float32),
            jax.ShapeDtypeStruct((n, nh), jnp.float32),
        ],
    )
    mid = pl.pallas_call(
        functools.partial(_mid_tc_body, n, nh),
        out_shape=[
            jax.ShapeDtypeStruct((n + 8, 2 * nh), jnp.float32),
            jax.ShapeDtypeStruct((n, nh), jnp.float32),
        ],
    )
    final = pl.pallas_call(
        functools.partial(_final_tc_body, n),
        out_shape=jax.ShapeDtypeStruct((G, Wl.shape[0]), jnp.float32),
    )

    n_pad = NT * nt_pad
    B1, C1 = first(x, W1, b1r)
    (S1,) = seg(csrc, cdst, counts, B1)
    B2, C2 = mid(C1, S1.reshape(n_pad, nh), g1r, be1r, W2, b2r)
    (S2,) = seg(csrc, cdst, counts, B2)
    B3, C3 = mid(C2, S2.reshape(n_pad, nh), g2r, be2r, W3, b3r)
    (S3,) = seg(csrc, cdst, counts, B3)
    return final(C3, S3.reshape(n_pad, nh), g3r, be3r, batch2, Wl, blr)
